# baseline (speedup 1.0000x reference)
import functools

import jax
import jax.numpy as jnp
from jax import lax
from jax.experimental import pallas as pl
from jax.experimental.pallas import tpu as pltpu
from jax.experimental.pallas import tpu_sc as plsc

_N = 8192
_M = 8192
_D = 128
_ROWS = 256
_G = _N // _ROWS

_NC = 2
_NS = 16
_NW = _NC * _NS
_RPW = _N // _NW
_ZROWS = 8
_NCHUNK = _RPW // _ZROWS


def _tc_body(xd_ref, yd_ref, mask_ref, pairs_ref, topd_ref, dscr):
    i = pl.program_id(0)

    @pl.when(i == 0)
    def _():
        diff = xd_ref[0] - yd_ref[0]
        dscr[0, 0] = jnp.sqrt(jnp.sum(diff * diff))

    d = dscr[0, 0]
    dm = mask_ref[0] * d
    rmin = jnp.min(dm, axis=1, keepdims=True)
    iota = jax.lax.broadcasted_iota(jnp.int32, (_ROWS, _M), 1)
    cand = jnp.where(dm == rmin, iota, jnp.int32(_M))
    argm = jnp.min(cand, axis=1, keepdims=True)
    topd_ref[0, 0] = rmin[:, 0]
    pairs_ref[0, 0] = argm[:, 0]


def _tc_reduce(xd, yd, mask):
    return pl.pallas_call(
        _tc_body,
        grid=(_G,),
        in_specs=[
            pl.BlockSpec((1, _N, _D), lambda i: (0, 0, 0)),
            pl.BlockSpec((1, _M, _D), lambda i: (0, 0, 0)),
            pl.BlockSpec((1, _ROWS, _M), lambda i: (0, i, 0)),
        ],
        out_specs=[
            pl.BlockSpec((1, 1, _ROWS), lambda i: (i, 0, 0)),
            pl.BlockSpec((1, 1, _ROWS), lambda i: (i, 0, 0)),
        ],
        out_shape=[
            jax.ShapeDtypeStruct((_G, 1, _ROWS), jnp.int32),
            jax.ShapeDtypeStruct((_G, 1, _ROWS), jnp.float32),
        ],
        scratch_shapes=[pltpu.SMEM((1, 1), jnp.float32)],
    )(xd, yd, mask)


_SC_MESH = plsc.VectorSubcoreMesh(core_axis_name="c", subcore_axis_name="s")


@functools.partial(
    pl.kernel,
    out_type=jax.ShapeDtypeStruct((_N, _M), jnp.float32),
    mesh=_SC_MESH,
    scratch_types=[
        pltpu.VMEM((_ZROWS, _M), jnp.float32),
        pltpu.SemaphoreType.DMA,
    ],
)
def _sc_zeros(out_hbm, zbuf, sem):
    wid = lax.axis_index("s") * _NC + lax.axis_index("c")
    zero16 = jnp.zeros((16,), jnp.float32)

    def zinit(c, _):
        for r in range(_ZROWS):
            zbuf[r, pl.ds(c * 16, 16)] = zero16
        return 0

    lax.fori_loop(0, _M // 16, zinit, 0)

    base = wid * _RPW

    def issue(i, _):
        pltpu.async_copy(zbuf, out_hbm.at[pl.ds(base + i * _ZROWS, _ZROWS)], sem)
        return 0

    lax.fori_loop(0, _NCHUNK, issue, 0)

    def drain(i, _):
        pltpu.make_async_copy(zbuf, out_hbm.at[pl.ds(base, _ZROWS)], sem).wait()
        return 0

    lax.fori_loop(0, _NCHUNK, drain, 0)


@functools.partial(
    pl.kernel,
    out_type=(),
    mesh=_SC_MESH,
    scratch_types=[
        pltpu.VMEM((_RPW,), jnp.int32),
        pltpu.VMEM((128,), jnp.int32),
        pltpu.VMEM((128,), jnp.int32),
        pltpu.VMEM((128,), jnp.float32),
        pltpu.SemaphoreType.DMA,
    ],
)
def _sc_ones(zflat_ref, pairs_hbm, pv, idx_a, idx_b, ones_v, sem):
    wid = lax.axis_index("s") * _NC + lax.axis_index("c")
    base = wid * _RPW
    pltpu.sync_copy(pairs_hbm.at[pl.ds(base, _RPW)], pv)
    iota = lax.iota(jnp.int32, 16)
    one16 = jnp.ones((16,), jnp.float32)
    for c in range(8):
        ones_v[pl.ds(c * 16, 16)] = one16
    for c in range(8):
        p16 = pv[pl.ds(c * 16, 16)]
        idx_a[pl.ds(c * 16, 16)] = (base + c * 16 + iota) * _M + p16
    for c in range(8):
        p16 = pv[pl.ds(128 + c * 16, 16)]
        idx_b[pl.ds(c * 16, 16)] = (base + 128 + c * 16 + iota) * _M + p16
    pltpu.async_copy(ones_v, zflat_ref.at[idx_a], sem).wait()
    pltpu.async_copy(ones_v, zflat_ref.at[idx_b], sem).wait()


def kernel(xd, yd, mask):
    pairs3, topd3 = _tc_reduce(xd, yd, mask)
    pairs = pairs3.reshape(1, _N)
    top_dists = topd3.reshape(1, _N)
    z = _sc_zeros()
    zref = jax.new_ref(z.reshape(_N * _M))
    _sc_ones(zref, pairs3.reshape(_N))
    new_mask = zref[...].reshape(1, _N, _M)
    return new_mask, pairs, top_dists


# speedup vs baseline: 1.3827x; 1.3827x over previous
import functools

import jax
import jax.numpy as jnp
from jax import lax
from jax.experimental import pallas as pl
from jax.experimental.pallas import tpu as pltpu
from jax.experimental.pallas import tpu_sc as plsc

_N = 8192
_M = 8192
_D = 128
_ROWS = 256
_G = _N // _ROWS

_NC = 2
_NS = 16
_NW = _NC * _NS
_RPW = _N // _NW
_ZROWS = 8
_NCHUNK = _RPW // _ZROWS


def _tc_body(xd_ref, yd_ref, mask_ref, pairs_ref, topd_ref, dscr):
    i = pl.program_id(0)

    @pl.when(i == 0)
    def _():
        diff = xd_ref[0] - yd_ref[0]
        dscr[0, 0] = jnp.sqrt(jnp.sum(diff * diff))

    d = dscr[0, 0]
    dm = mask_ref[0] * d
    rmin = jnp.min(dm, axis=1, keepdims=True)
    iota = jax.lax.broadcasted_iota(jnp.int32, (_ROWS, _M), 1)
    cand = jnp.where(dm == rmin, iota, jnp.int32(_M))
    argm = jnp.min(cand, axis=1, keepdims=True)
    topd_ref[0, 0] = rmin[:, 0]
    pairs_ref[0, 0] = argm[:, 0]


def _tc_reduce(xd, yd, mask):
    return pl.pallas_call(
        _tc_body,
        grid=(_G,),
        in_specs=[
            pl.BlockSpec((1, _N, _D), lambda i: (0, 0, 0)),
            pl.BlockSpec((1, _M, _D), lambda i: (0, 0, 0)),
            pl.BlockSpec((1, _ROWS, _M), lambda i: (0, i, 0)),
        ],
        out_specs=[
            pl.BlockSpec((1, 1, _ROWS), lambda i: (i, 0, 0)),
            pl.BlockSpec((1, 1, _ROWS), lambda i: (i, 0, 0)),
        ],
        out_shape=[
            jax.ShapeDtypeStruct((_G, 1, _ROWS), jnp.int32),
            jax.ShapeDtypeStruct((_G, 1, _ROWS), jnp.float32),
        ],
        scratch_shapes=[pltpu.SMEM((1, 1), jnp.float32)],
    )(xd, yd, mask)


_SC_MESH = plsc.VectorSubcoreMesh(core_axis_name="c", subcore_axis_name="s")


_ZCHUNK = _ZROWS * _M


@functools.partial(
    pl.kernel,
    out_type=jax.ShapeDtypeStruct((_N * _M,), jnp.float32),
    mesh=_SC_MESH,
    scratch_types=[
        pltpu.VMEM((_ZCHUNK,), jnp.float32),
        pltpu.SemaphoreType.DMA,
    ],
)
def _sc_zeros(out_hbm, zbuf, sem):
    wid = lax.axis_index("s") * _NC + lax.axis_index("c")
    zero16 = jnp.zeros((16,), jnp.float32)

    def zinit(c, _):
        for r in range(8):
            zbuf[pl.ds(c * 128 + r * 16, 16)] = zero16
        return 0

    lax.fori_loop(0, _ZCHUNK // 128, zinit, 0)

    base = wid * _RPW * _M

    def issue(i, _):
        pltpu.async_copy(zbuf, out_hbm.at[pl.ds(base + i * _ZCHUNK, _ZCHUNK)], sem)
        return 0

    lax.fori_loop(0, _NCHUNK, issue, 0)

    def drain(i, _):
        pltpu.make_async_copy(zbuf, out_hbm.at[pl.ds(base, _ZCHUNK)], sem).wait()
        return 0

    lax.fori_loop(0, _NCHUNK, drain, 0)


@functools.partial(
    pl.kernel,
    out_type=(),
    mesh=_SC_MESH,
    scratch_types=[
        pltpu.VMEM((_RPW,), jnp.int32),
        pltpu.VMEM((128,), jnp.int32),
        pltpu.VMEM((128,), jnp.int32),
        pltpu.VMEM((128,), jnp.float32),
        pltpu.SemaphoreType.DMA,
    ],
)
def _sc_ones(zflat_ref, pairs_hbm, pv, idx_a, idx_b, ones_v, sem):
    wid = lax.axis_index("s") * _NC + lax.axis_index("c")
    base = wid * _RPW
    pltpu.sync_copy(pairs_hbm.at[pl.ds(base, _RPW)], pv)
    iota = lax.iota(jnp.int32, 16)
    one16 = jnp.ones((16,), jnp.float32)
    for c in range(8):
        ones_v[pl.ds(c * 16, 16)] = one16
    for c in range(8):
        p16 = pv[pl.ds(c * 16, 16)]
        idx_a[pl.ds(c * 16, 16)] = (base + c * 16 + iota) * _M + p16
    for c in range(8):
        p16 = pv[pl.ds(128 + c * 16, 16)]
        idx_b[pl.ds(c * 16, 16)] = (base + 128 + c * 16 + iota) * _M + p16
    pltpu.async_copy(ones_v, zflat_ref.at[idx_a], sem).wait()
    pltpu.async_copy(ones_v, zflat_ref.at[idx_b], sem).wait()


def kernel(xd, yd, mask):
    pairs3, topd3 = _tc_reduce(xd, yd, mask)
    pairs = pairs3.reshape(1, _N)
    top_dists = topd3.reshape(1, _N)
    z = _sc_zeros()
    zref = jax.new_ref(z)
    _sc_ones(zref, pairs3.reshape(_N))
    new_mask = jax.freeze(zref).reshape(1, _N, _M)
    return new_mask, pairs, top_dists


# speedup vs baseline: 3.2706x; 2.3653x over previous
import functools

import jax
import jax.numpy as jnp
from jax import lax
from jax.experimental import pallas as pl
from jax.experimental.pallas import tpu as pltpu
from jax.experimental.pallas import tpu_sc as plsc

_N = 8192
_M = 8192
_D = 128
_ROWS = 256
_G = _N // _ROWS

_NC = 2
_NS = 16
_NW = _NC * _NS
_RPW = _N // _NW


def _tc_body(xd_ref, yd_ref, mask_ref, pairs_ref, topd_ref, dscr):
    i = pl.program_id(0)

    @pl.when(i == 0)
    def _():
        diff = xd_ref[0] - yd_ref[0]
        dscr[0, 0] = jnp.sqrt(jnp.sum(diff * diff))

    d = dscr[0, 0]
    dm = mask_ref[0] * d
    rmin = jnp.min(dm, axis=1, keepdims=True)
    iota = jax.lax.broadcasted_iota(jnp.int32, (_ROWS, _M), 1)
    cand = jnp.where(dm == rmin, iota, jnp.int32(_M))
    argm = jnp.min(cand, axis=1, keepdims=True)
    topd_ref[0, 0] = rmin[:, 0]
    pairs_ref[0, 0] = argm[:, 0]


def _tc_reduce(xd, yd, mask):
    return pl.pallas_call(
        _tc_body,
        grid=(_G,),
        in_specs=[
            pl.BlockSpec((1, _N, _D), lambda i: (0, 0, 0)),
            pl.BlockSpec((1, _M, _D), lambda i: (0, 0, 0)),
            pl.BlockSpec((1, _ROWS, _M), lambda i: (0, i, 0)),
        ],
        out_specs=[
            pl.BlockSpec((1, 1, _ROWS), lambda i: (i, 0, 0)),
            pl.BlockSpec((1, 1, _ROWS), lambda i: (i, 0, 0)),
        ],
        out_shape=[
            jax.ShapeDtypeStruct((_G, 1, _ROWS), jnp.int32),
            jax.ShapeDtypeStruct((_G, 1, _ROWS), jnp.float32),
        ],
        scratch_shapes=[pltpu.SMEM((1, 1), jnp.float32)],
    )(xd, yd, mask)


_SC_MESH = plsc.VectorSubcoreMesh(core_axis_name="c", subcore_axis_name="s")


@functools.partial(
    pl.kernel,
    out_type=jax.ShapeDtypeStruct((1, _N, _M), jnp.float32),
    mesh=_SC_MESH,
    scratch_types=[
        pltpu.VMEM((_RPW + 16,), jnp.int32),
        pltpu.VMEM((_M,), jnp.float32),
        pltpu.VMEM((_M,), jnp.float32),
        pltpu.SemaphoreType.DMA,
        pltpu.SemaphoreType.DMA,
    ],
)
def _sc_onehot(pairs_hbm, out_hbm, pv, buf_a, buf_b, sem_a, sem_b):
    wid = lax.axis_index("s") * _NC + lax.axis_index("c")
    base = wid * _RPW
    pltpu.sync_copy(pairs_hbm.at[pl.ds(base, _RPW)], pv.at[pl.ds(0, _RPW)])

    zero16 = jnp.zeros((16,), jnp.float32)
    iota = lax.iota(jnp.int32, 16)

    def bzero(c, _):
        buf_a[pl.ds(c * 16, 16)] = zero16
        buf_b[pl.ds(c * 16, 16)] = zero16
        return 0

    lax.fori_loop(0, _M // 16, bzero, 0)

    def hot_chunk(r):
        col = pv[pl.ds(r, 16)][0]
        coff = jnp.bitwise_and(col, jnp.int32(-16))
        lane = jnp.bitwise_and(col, jnp.int32(15))
        h16 = jnp.where(iota == lane, jnp.float32(1.0), jnp.float32(0.0))
        return coff, h16

    off_a, h = hot_chunk(0)
    buf_a[pl.ds(off_a, 16)] = h
    pltpu.async_copy(buf_a, out_hbm.at[0, base], sem_a)
    off_b, h = hot_chunk(1)
    buf_b[pl.ds(off_b, 16)] = h
    pltpu.async_copy(buf_b, out_hbm.at[0, base + 1], sem_b)

    def body(j, carry):
        off_a, off_b = carry
        r0 = 2 * j
        pltpu.make_async_copy(buf_a, out_hbm.at[0, base], sem_a).wait()
        buf_a[pl.ds(off_a, 16)] = zero16
        off_a, h = hot_chunk(r0)
        buf_a[pl.ds(off_a, 16)] = h
        pltpu.async_copy(buf_a, out_hbm.at[0, base + r0], sem_a)

        pltpu.make_async_copy(buf_b, out_hbm.at[0, base], sem_b).wait()
        buf_b[pl.ds(off_b, 16)] = zero16
        off_b, h = hot_chunk(r0 + 1)
        buf_b[pl.ds(off_b, 16)] = h
        pltpu.async_copy(buf_b, out_hbm.at[0, base + r0 + 1], sem_b)
        return off_a, off_b

    lax.fori_loop(1, _RPW // 2, body, (off_a, off_b))
    pltpu.make_async_copy(buf_a, out_hbm.at[0, base], sem_a).wait()
    pltpu.make_async_copy(buf_b, out_hbm.at[0, base], sem_b).wait()


def kernel(xd, yd, mask):
    pairs3, topd3 = _tc_reduce(xd, yd, mask)
    pairs = pairs3.reshape(1, _N)
    top_dists = topd3.reshape(1, _N)
    new_mask = _sc_onehot(pairs3.reshape(_N))
    return new_mask, pairs, top_dists


# speedup vs baseline: 3.4160x; 1.0445x over previous
import functools

import jax
import jax.numpy as jnp
from jax import lax
from jax.experimental import pallas as pl
from jax.experimental.pallas import tpu as pltpu
from jax.experimental.pallas import tpu_sc as plsc

_N = 8192
_M = 8192
_D = 128
_ROWS = 512
_G = _N // _ROWS

_NC = 2
_NS = 16
_NW = _NC * _NS
_RPW = _N // _NW


def _tc_body(xd_ref, yd_ref, mask_ref, pairs_ref, topd_ref, dscr):
    i = pl.program_id(0)

    @pl.when(i == 0)
    def _():
        diff = xd_ref[0] - yd_ref[0]
        dscr[0, 0] = jnp.sqrt(jnp.sum(diff * diff))

    d = dscr[0, 0]
    dm = mask_ref[0] * d
    rmin = jnp.min(dm, axis=1, keepdims=True)
    iota = jax.lax.broadcasted_iota(jnp.int32, (_ROWS, _M), 1)
    cand = jnp.where(dm == rmin, iota, jnp.int32(_M))
    argm = jnp.min(cand, axis=1, keepdims=True)
    topd_ref[0, 0] = rmin[:, 0]
    pairs_ref[0, 0] = argm[:, 0]


def _tc_reduce(xd, yd, mask):
    return pl.pallas_call(
        _tc_body,
        grid=(_G,),
        in_specs=[
            pl.BlockSpec((1, _N, _D), lambda i: (0, 0, 0)),
            pl.BlockSpec((1, _M, _D), lambda i: (0, 0, 0)),
            pl.BlockSpec((1, _ROWS, _M), lambda i: (0, i, 0)),
        ],
        out_specs=[
            pl.BlockSpec((1, 1, _ROWS), lambda i: (i, 0, 0)),
            pl.BlockSpec((1, 1, _ROWS), lambda i: (i, 0, 0)),
        ],
        out_shape=[
            jax.ShapeDtypeStruct((_G, 1, _ROWS), jnp.int32),
            jax.ShapeDtypeStruct((_G, 1, _ROWS), jnp.float32),
        ],
        scratch_shapes=[pltpu.SMEM((1, 1), jnp.float32)],
    )(xd, yd, mask)


_SC_MESH = plsc.VectorSubcoreMesh(core_axis_name="c", subcore_axis_name="s")


@functools.partial(
    pl.kernel,
    out_type=jax.ShapeDtypeStruct((1, _N, _M), jnp.float32),
    mesh=_SC_MESH,
    scratch_types=[
        pltpu.VMEM((_RPW + 16,), jnp.int32),
        pltpu.VMEM((_M,), jnp.float32),
        pltpu.VMEM((_M,), jnp.float32),
        pltpu.SemaphoreType.DMA,
        pltpu.SemaphoreType.DMA,
    ],
)
def _sc_onehot(pairs_hbm, out_hbm, pv, buf_a, buf_b, sem_a, sem_b):
    wid = lax.axis_index("s") * _NC + lax.axis_index("c")
    base = wid * _RPW
    pltpu.sync_copy(pairs_hbm.at[pl.ds(base, _RPW)], pv.at[pl.ds(0, _RPW)])

    zero16 = jnp.zeros((16,), jnp.float32)
    iota = lax.iota(jnp.int32, 16)

    def bzero(c, _):
        buf_a[pl.ds(c * 16, 16)] = zero16
        buf_b[pl.ds(c * 16, 16)] = zero16
        return 0

    lax.fori_loop(0, _M // 16, bzero, 0)

    def hot_chunk(r):
        col = pv[pl.ds(r, 16)][0]
        coff = jnp.bitwise_and(col, jnp.int32(-16))
        lane = jnp.bitwise_and(col, jnp.int32(15))
        h16 = jnp.where(iota == lane, jnp.float32(1.0), jnp.float32(0.0))
        return coff, h16

    off_a, h = hot_chunk(0)
    buf_a[pl.ds(off_a, 16)] = h
    pltpu.async_copy(buf_a, out_hbm.at[0, base], sem_a)
    off_b, h = hot_chunk(1)
    buf_b[pl.ds(off_b, 16)] = h
    pltpu.async_copy(buf_b, out_hbm.at[0, base + 1], sem_b)

    def body(j, carry):
        off_a, off_b = carry
        r0 = 2 * j
        pltpu.make_async_copy(buf_a, out_hbm.at[0, base], sem_a).wait()
        buf_a[pl.ds(off_a, 16)] = zero16
        off_a, h = hot_chunk(r0)
        buf_a[pl.ds(off_a, 16)] = h
        pltpu.async_copy(buf_a, out_hbm.at[0, base + r0], sem_a)

        pltpu.make_async_copy(buf_b, out_hbm.at[0, base], sem_b).wait()
        buf_b[pl.ds(off_b, 16)] = zero16
        off_b, h = hot_chunk(r0 + 1)
        buf_b[pl.ds(off_b, 16)] = h
        pltpu.async_copy(buf_b, out_hbm.at[0, base + r0 + 1], sem_b)
        return off_a, off_b

    lax.fori_loop(1, _RPW // 2, body, (off_a, off_b))
    pltpu.make_async_copy(buf_a, out_hbm.at[0, base], sem_a).wait()
    pltpu.make_async_copy(buf_b, out_hbm.at[0, base], sem_b).wait()


def kernel(xd, yd, mask):
    pairs3, topd3 = _tc_reduce(xd, yd, mask)
    pairs = pairs3.reshape(1, _N)
    top_dists = topd3.reshape(1, _N)
    new_mask = _sc_onehot(pairs3.reshape(_N))
    return new_mask, pairs, top_dists
